# Initial kernel scaffold; baseline (speedup 1.0000x reference)
import jax
import jax.numpy as jnp
from jax.experimental import pallas as pl


def kernel(x, edge_index, W1l, b1, W1r, W2l, b2, W2r, Wh, bh):
    raise NotImplementedError("write your pallas kernel here")


# speedup vs baseline: 14.7932x; 14.7932x over previous
import functools

import jax
import jax.numpy as jnp
from jax import lax
from jax.experimental import pallas as pl
from jax.experimental.pallas import tpu as pltpu
from jax.experimental.pallas import tpu_sc as plsc

N = 10000
E = 320000
D = 128
H = 64
H2 = 32
F1 = 80

NC = 2
NS = 16
NW = NC * NS
EW = E // NW
C = 125
NCHUNK = EW // C
NP = 10240
RPS = NP // NS


def _make_sc_agg(F: int):
  mesh = plsc.VectorSubcoreMesh(core_axis_name="c", subcore_axis_name="s")

  @functools.partial(
      pl.kernel,
      mesh=mesh,
      compiler_params=pltpu.CompilerParams(use_tc_tiling_on_sc=False),
      out_type=jax.ShapeDtypeStruct((NC, NP, F), jnp.float32),
      scratch_types=[
          pltpu.VMEM((NCHUNK, C), jnp.int32),
          pltpu.VMEM((NCHUNK, C), jnp.int32),
          pltpu.VMEM((C, F), jnp.float32),
          pltpu.VMEM((C, F), jnp.float32),
          pltpu.VMEM((C, F), jnp.float32),
          pltpu.VMEM((C, F), jnp.float32),
          pltpu.VMEM_SHARED((NP, F), jnp.float32),
          pltpu.SemaphoreType.DMA,
          pltpu.SemaphoreType.DMA,
          pltpu.SemaphoreType.DMA,
          pltpu.SemaphoreType.DMA,
      ],
  )
  def sc_agg(y_hbm, src_hbm, dst_hbm, zero_hbm, out_hbm,
             src_w, dst_w, buf0, buf1, buf2, buf3, acc_sh,
             sem0, sem1, sem2, sem3):
    cid = lax.axis_index("c")
    sid = lax.axis_index("s")
    wid = cid * NS + sid
    bufs = (buf0, buf1, buf2, buf3)
    sems = (sem0, sem1, sem2, sem3)

    pltpu.sync_copy(src_hbm.at[wid], src_w)
    pltpu.sync_copy(dst_hbm.at[wid], dst_w)

    rbase = sid * RPS
    pltpu.sync_copy(zero_hbm.at[pl.ds(rbase, RPS)], acc_sh.at[pl.ds(rbase, RPS)])
    plsc.subcore_barrier()

    def body(j, carry):
      i = 4 * j
      handles = [pltpu.async_copy(y_hbm.at[src_w.at[i + k]], bufs[k], sems[k])
                 for k in range(4)]
      for k in range(4):
        handles[k].wait()
        pltpu.sync_copy(bufs[k], acc_sh.at[dst_w.at[i + k]], add=True)
      return carry

    lax.fori_loop(0, NCHUNK // 4, body, 0)
    plsc.subcore_barrier()

    pltpu.sync_copy(acc_sh.at[pl.ds(rbase, RPS)],
                    out_hbm.at[cid, pl.ds(rbase, RPS)])

  return sc_agg


_sc_agg_f1 = _make_sc_agg(F1)
_sc_agg_h2 = _make_sc_agg(H2)


def _tc1_body(x_ref, wl_ref, wr_ref, b_ref, y_ref, z_ref):
  x = x_ref[...]
  yl = jnp.dot(x, wl_ref[...], preferred_element_type=jnp.float32)
  col = lax.broadcasted_iota(jnp.int32, (1, F1), 1)
  y_ref[...] = yl + jnp.where(col == H, 1.0, 0.0)
  z_ref[...] = jnp.dot(x, wr_ref[...], preferred_element_type=jnp.float32) + b_ref[...]


_tc1 = pl.pallas_call(
    _tc1_body,
    out_shape=[jax.ShapeDtypeStruct((N, F1), jnp.float32),
               jax.ShapeDtypeStruct((N, H), jnp.float32)],
)


def _tc2_body(p_ref, z1_ref, wl_ref, wr_ref, b_ref, y2_ref, z2_ref, c_ref):
  agg = p_ref[0, :N] + p_ref[1, :N]
  c = jnp.maximum(agg[:, H:H + 1], 1.0)
  h = jnp.maximum(agg[:, :H] / c + z1_ref[...], 0.0)
  y2_ref[...] = jnp.dot(h, wl_ref[...], preferred_element_type=jnp.float32)
  z2_ref[...] = jnp.dot(h, wr_ref[...], preferred_element_type=jnp.float32) + b_ref[...]
  c_ref[...] = c


_tc2 = pl.pallas_call(
    _tc2_body,
    out_shape=[jax.ShapeDtypeStruct((N, H2), jnp.float32),
               jax.ShapeDtypeStruct((N, H2), jnp.float32),
               jax.ShapeDtypeStruct((N, 1), jnp.float32)],
)


def _tc3_body(p_ref, z2_ref, c_ref, wh_ref, bh_ref, o_ref):
  agg = p_ref[0, :N] + p_ref[1, :N]
  h = jnp.maximum(agg / c_ref[...] + z2_ref[...], 0.0)
  o_ref[...] = jnp.dot(h, wh_ref[...], preferred_element_type=jnp.float32) + bh_ref[...]


_tc3 = pl.pallas_call(
    _tc3_body,
    out_shape=jax.ShapeDtypeStruct((N, 1), jnp.float32),
)


def kernel(x, edge_index, W1l, b1, W1r, W2l, b2, W2r, Wh, bh):
  src3 = edge_index[0].reshape(NW, NCHUNK, C)
  dst3 = edge_index[1].reshape(NW, NCHUNK, C)
  w1lp = jnp.pad(W1l, ((0, 0), (0, F1 - H)))

  y1, z1 = _tc1(x, w1lp, W1r, b1.reshape(1, H))
  p1 = _sc_agg_f1(y1, src3, dst3, jnp.zeros((NP, F1), jnp.float32))
  y2, z2, c = _tc2(p1, z1, W2l, W2r, b2.reshape(1, H2))
  p2 = _sc_agg_h2(y2, src3, dst3, jnp.zeros((NP, H2), jnp.float32))
  out = _tc3(p2, z2, c, Wh, bh.reshape(1, 1))
  return out[:, 0]
